# Initial kernel scaffold; baseline (speedup 1.0000x reference)
import jax
import jax.numpy as jnp
from jax.experimental import pallas as pl


def kernel(features, coin_id, regime_id, account, temporal, params):
    raise NotImplementedError("write your pallas kernel here")


# speedup vs baseline: 2.1111x; 2.1111x over previous
import functools

import jax
import jax.numpy as jnp
from jax import lax
from jax.experimental import pallas as pl

B = 4096
BLK = 256
N_COINS = 250
COIN_DIM = 32
REG_DIM = 16
N_ACC = 4
N_TEMP = 40
EH = 256
EO = 128
NE = 8
FUSION = 256
NLAB = 8
NGROUPS = 4
FEAT_DIM = 256
PART_SLICES = ((0, 64), (64, 128), (128, 192), (192, 256))
PART_NAMES = ("price", "volume", "orderflow", "derived")

_SQRT2 = 1.4142135623730951
_RSQRT_EO = 1.0 / (EO ** 0.5)


def _gelu(x):
    return 0.5 * x * (1.0 + lax.erf(x / _SQRT2))


def _ln(x, g, b, eps=1e-5):
    m = jnp.mean(x, axis=-1, keepdims=True)
    xc = x - m
    v = jnp.mean(xc * xc, axis=-1, keepdims=True)
    return xc * lax.rsqrt(v + eps) * g + b


def _dot(x, w):
    return jnp.dot(x, w, preferred_element_type=jnp.float32)


def _pack_weights(p):
    f32 = jnp.float32
    w = {}
    coin = jnp.zeros((256, COIN_DIM), f32).at[:N_COINS].set(p["coin_emb"])
    w["coin_emb"] = coin
    reg = jnp.zeros((128, REG_DIM), f32).at[:4].set(p["regime_emb"])
    w["regime_emb"] = reg
    w["temp1_w"] = p["temp1"]["w"]
    w["temp1_b"] = p["temp1"]["b"][None]
    w["temp2_w"] = p["temp2"]["w"]
    w["temp2_b"] = p["temp2"]["b"][None]
    w["temp_lng"] = p["temp_lng"][None]
    w["temp_lnb"] = p["temp_lnb"][None]

    w1f, wrf, w2s, w3s = [], [], [], []
    b1s, b2s, b3s, brs, lgs, lbs = [], [], [], [], [], []
    for name, (a, b) in zip(PART_NAMES, PART_SLICES):
        ep = p["feat_experts"][name]
        w1f.append(jnp.zeros((FEAT_DIM, EH), f32).at[a:b].set(ep["w1"]))
        wrf.append(jnp.zeros((FEAT_DIM, EO), f32).at[a:b].set(ep["wr"]))
        w2s.append(ep["w2"])
        w3s.append(ep["w3"])
        b1s.append(ep["b1"][None])
        b2s.append(ep["b2"][None])
        b3s.append(ep["b3"][None])
        brs.append(ep["br"][None])
        lgs.append(ep["lng"][None])
        lbs.append(ep["lnb"][None])
    w["fe_w1"] = jnp.stack(w1f)
    w["fe_wr"] = jnp.stack(wrf)
    w["fe_w2"] = jnp.stack(w2s)
    w["fe_w3"] = jnp.stack(w3s)
    w["fe_b1"] = jnp.stack(b1s)
    w["fe_b2"] = jnp.stack(b2s)
    w["fe_b3"] = jnp.stack(b3s)
    w["fe_br"] = jnp.stack(brs)
    w["fe_lng"] = jnp.stack(lgs)
    w["fe_lnb"] = jnp.stack(lbs)

    cw = p["context"]["w"]
    w["ctx_wa"] = cw[0:N_ACC]
    w["ctx_wc"] = cw[N_ACC:N_ACC + COIN_DIM]
    w["ctx_wr"] = cw[N_ACC + COIN_DIM:N_ACC + COIN_DIM + REG_DIM]
    w["ctx_wt"] = cw[N_ACC + COIN_DIM + REG_DIM:]
    w["ctx_b"] = p["context"]["b"][None]

    qw = p["gate_q"]["w"]
    w["gq_parts"] = jnp.stack([qw[i * EO:(i + 1) * EO] for i in range(4)])
    w["gq_ctx"] = qw[4 * EO:]
    w["gq_b"] = p["gate_q"]["b"][None]
    w["gk_w"] = jnp.stack([p["gate_keys"][n]["w"] for n in PART_NAMES])
    w["gk_b"] = jnp.stack([p["gate_keys"][n]["b"][None] for n in PART_NAMES])

    rw = p["router1"]["w"]
    w["r1_g"] = rw[:EO]
    w["r1_r"] = rw[EO:]
    w["r1_b"] = p["router1"]["b"][None]
    w["r2_w"] = p["router2"]["w"]
    w["r2_b"] = p["router2"]["b"][None]

    w["moe_w1"] = jnp.stack([e["w1"] for e in p["moe_experts"]])
    w["moe_w2"] = jnp.stack([e["w2"] for e in p["moe_experts"]])
    w["moe_w3"] = jnp.stack([e["w3"] for e in p["moe_experts"]])
    w["moe_b1"] = jnp.stack([e["b1"][None] for e in p["moe_experts"]])
    w["moe_b2"] = jnp.stack([e["b2"][None] for e in p["moe_experts"]])
    w["moe_b3"] = jnp.stack([e["b3"][None] for e in p["moe_experts"]])
    w["moe_lng"] = jnp.stack([e["lng"][None] for e in p["moe_experts"]])
    w["moe_lnb"] = jnp.stack([e["lnb"][None] for e in p["moe_experts"]])

    fw = p["fus1"]["w"]
    w["f1_m"] = fw[:EO]
    w["f1_c"] = fw[EO:]
    w["f1_b"] = p["fus1"]["b"][None]
    w["f_ln1g"] = p["fus_ln1g"][None]
    w["f_ln1b"] = p["fus_ln1b"][None]
    w["f2_w"] = p["fus2"]["w"]
    w["f2_b"] = p["fus2"]["b"][None]
    w["f_ln2g"] = p["fus_ln2g"][None]
    w["f_ln2b"] = p["fus_ln2b"][None]

    for hname in ("lab", "mae", "mfe"):
        for lyr in ("1", "2"):
            key = hname + lyr
            w[f"hd_{key}_w"] = jnp.stack([h[key]["w"] for h in p["heads"]])
            w[f"hd_{key}_b"] = jnp.stack([h[key]["b"][None] for h in p["heads"]])
    w["conf1_w"] = p["conf1"]["w"]
    w["conf1_b"] = p["conf1"]["b"][None]
    w["conf2_w"] = p["conf2"]["w"]
    w["conf2_b"] = p["conf2"]["b"][None]
    w["lev1_w"] = p["lev1"]["w"]
    w["lev1_b"] = p["lev1"]["b"][None]
    w["lev2_w"] = p["lev2"]["w"]
    w["lev2_b"] = p["lev2"]["b"][None]
    return w


_WNAMES = None


def _body(names, *refs):
    feats_ref, coin_ref, reg_ref, acct_ref, temp_ref = refs[:5]
    out_ref = refs[-1]
    w = {n: r for n, r in zip(names, refs[5:-1])}

    feats = feats_ref[...]
    coin_id = coin_ref[...]
    regime_id = reg_ref[...]
    acct = acct_ref[...]
    temporal = temp_ref[...]

    iota_c = lax.broadcasted_iota(jnp.int32, (BLK, 256), 1)
    oh_c = (iota_c == coin_id).astype(jnp.float32)
    coin_emb = _dot(oh_c, w["coin_emb"][...])
    iota_r = lax.broadcasted_iota(jnp.int32, (BLK, 128), 1)
    oh_r = (iota_r == regime_id).astype(jnp.float32)
    regime_emb = _dot(oh_r, w["regime_emb"][...])

    t = _gelu(_dot(temporal, w["temp1_w"][...]) + w["temp1_b"][...])
    t = _dot(t, w["temp2_w"][...]) + w["temp2_b"][...]
    temporal_enc = _ln(t, w["temp_lng"][...], w["temp_lnb"][...])

    feat_outs = []
    for i in range(4):
        h = _gelu(_dot(feats, w["fe_w1"][i]) + w["fe_b1"][i])
        h = _gelu(_dot(h, w["fe_w2"][i]) + w["fe_b2"][i])
        h = _dot(h, w["fe_w3"][i]) + w["fe_b3"][i]
        res = _dot(feats, w["fe_wr"][i]) + w["fe_br"][i]
        feat_outs.append(_ln(h + res, w["fe_lng"][i], w["fe_lnb"][i]))

    ctx = (_dot(acct, w["ctx_wa"][...]) + _dot(coin_emb, w["ctx_wc"][...])
           + _dot(regime_emb, w["ctx_wr"][...])
           + _dot(temporal_enc, w["ctx_wt"][...]) + w["ctx_b"][...])
    context_enc = _gelu(ctx)

    q = w["gq_b"][...] + _dot(context_enc, w["gq_ctx"][...])
    for i in range(4):
        q = q + _dot(feat_outs[i], w["gq_parts"][i])
    scores = []
    for i in range(4):
        k = _dot(feat_outs[i], w["gk_w"][i]) + w["gk_b"][i]
        scores.append(jnp.sum(q * k, axis=-1, keepdims=True) * _RSQRT_EO)
    smax = jnp.maximum(jnp.maximum(scores[0], scores[1]),
                       jnp.maximum(scores[2], scores[3]))
    exps = [jnp.exp(s - smax) for s in scores]
    denom = exps[0] + exps[1] + exps[2] + exps[3]
    gated = jnp.zeros((BLK, EO), jnp.float32)
    for i in range(4):
        gated = gated + (exps[i] / denom) * feat_outs[i]

    rh = _gelu(_dot(gated, w["r1_g"][...]) + _dot(regime_emb, w["r1_r"][...])
               + w["r1_b"][...])
    logits = _dot(rh, w["r2_w"][...]) + w["r2_b"][...]
    iota8 = lax.broadcasted_iota(jnp.int32, (BLK, NE), 1)
    m1 = jnp.max(logits, axis=-1, keepdims=True)
    i1 = jnp.min(jnp.where(logits == m1, iota8, NE), axis=-1, keepdims=True)
    masked = jnp.where(iota8 == i1, -1e30, logits)
    m2 = jnp.max(masked, axis=-1, keepdims=True)
    i2 = jnp.min(jnp.where(masked == m2, iota8, NE), axis=-1, keepdims=True)
    e2 = jnp.exp(m2 - m1)
    w1c = 1.0 / (1.0 + e2)
    w2c = e2 * w1c
    coefs = (jnp.where(iota8 == i1, w1c, 0.0)
             + jnp.where(iota8 == i2, w2c, 0.0))

    moe = jnp.zeros((BLK, EO), jnp.float32)
    for e in range(NE):
        h = _gelu(_dot(gated, w["moe_w1"][e]) + w["moe_b1"][e])
        h = _gelu(_dot(h, w["moe_w2"][e]) + w["moe_b2"][e])
        h = _dot(h, w["moe_w3"][e]) + w["moe_b3"][e]
        eo = _ln(h + gated, w["moe_lng"][e], w["moe_lnb"][e])
        moe = moe + lax.slice_in_dim(coefs, e, e + 1, axis=1) * eo

    f = _gelu(_dot(moe, w["f1_m"][...]) + _dot(context_enc, w["f1_c"][...])
              + w["f1_b"][...])
    f = _ln(f, w["f_ln1g"][...], w["f_ln1b"][...])
    f = _gelu(_dot(f, w["f2_w"][...]) + w["f2_b"][...])
    f = _ln(f, w["f_ln2g"][...], w["f_ln2b"][...])

    pieces = []
    for hname in ("lab", "mae", "mfe"):
        for g in range(NGROUPS):
            h1 = _gelu(_dot(f, w[f"hd_{hname}1_w"][g]) + w[f"hd_{hname}1_b"][g])
            pieces.append(_dot(h1, w[f"hd_{hname}2_w"][g]) + w[f"hd_{hname}2_b"][g])
    c = _gelu(_dot(f, w["conf1_w"][...]) + w["conf1_b"][...])
    pieces.append(jax.nn.sigmoid(_dot(c, w["conf2_w"][...]) + w["conf2_b"][...]))
    lv = _gelu(_dot(f, w["lev1_w"][...]) + w["lev1_b"][...])
    pieces.append(jax.nn.sigmoid(_dot(lv, w["lev2_w"][...]) + w["lev2_b"][...]))
    out_ref[...] = jnp.concatenate(pieces, axis=-1)


def _forward(features, coin_id, regime_id, account, temporal, params,
             interpret=False):
    w = _pack_weights(params)
    names = tuple(w.keys())
    warrs = [w[n] for n in names]
    coin2 = coin_id.astype(jnp.int32).reshape(B, 1)
    reg2 = regime_id.astype(jnp.int32).reshape(B, 1)

    def _const_spec(arr):
        nd = arr.ndim
        return pl.BlockSpec(arr.shape, lambda i, _nd=nd: (0,) * _nd)

    in_specs = [
        pl.BlockSpec((BLK, FEAT_DIM), lambda i: (i, 0)),
        pl.BlockSpec((BLK, 1), lambda i: (i, 0)),
        pl.BlockSpec((BLK, 1), lambda i: (i, 0)),
        pl.BlockSpec((BLK, N_ACC), lambda i: (i, 0)),
        pl.BlockSpec((BLK, N_TEMP), lambda i: (i, 0)),
    ] + [_const_spec(a) for a in warrs]

    out = pl.pallas_call(
        functools.partial(_body, names),
        grid=(B // BLK,),
        in_specs=in_specs,
        out_specs=pl.BlockSpec((BLK, 98), lambda i: (i, 0)),
        out_shape=jax.ShapeDtypeStruct((B, 98), jnp.float32),
        interpret=interpret,
    )(features, coin2, reg2, account, temporal, *warrs)
    return out


def kernel(features, coin_id, regime_id, account, temporal, params):
    return _forward(features, coin_id, regime_id, account, temporal, params)


# speedup vs baseline: 2.5462x; 1.2061x over previous
import functools

import jax
import jax.numpy as jnp
from jax import lax
from jax.experimental import pallas as pl

B = 4096
BLK = 512
N_COINS = 250
COIN_DIM = 32
REG_DIM = 16
N_ACC = 4
N_TEMP = 40
EH = 256
EO = 128
NE = 8
FUSION = 256
NLAB = 8
NGROUPS = 4
FEAT_DIM = 256
PART_SLICES = ((0, 64), (64, 128), (128, 192), (192, 256))
PART_NAMES = ("price", "volume", "orderflow", "derived")

_SQRT2 = 1.4142135623730951
_RSQRT_EO = 1.0 / (EO ** 0.5)


def _gelu(x):
    return 0.5 * x * (1.0 + lax.erf(x / _SQRT2))


def _ln(x, g, b, eps=1e-5):
    m = jnp.mean(x, axis=-1, keepdims=True)
    xc = x - m
    v = jnp.mean(xc * xc, axis=-1, keepdims=True)
    return xc * lax.rsqrt(v + eps) * g + b


def _dot(x, w):
    return jnp.dot(x, w, preferred_element_type=jnp.float32)


def _pack_weights(p):
    f32 = jnp.float32
    w = {}
    coin = jnp.zeros((256, COIN_DIM), f32).at[:N_COINS].set(p["coin_emb"])
    w["coin_emb"] = coin
    reg = jnp.zeros((128, REG_DIM), f32).at[:4].set(p["regime_emb"])
    w["regime_emb"] = reg
    w["temp1_w"] = p["temp1"]["w"]
    w["temp1_b"] = p["temp1"]["b"][None]
    w["temp2_w"] = p["temp2"]["w"]
    w["temp2_b"] = p["temp2"]["b"][None]
    w["temp_lng"] = p["temp_lng"][None]
    w["temp_lnb"] = p["temp_lnb"][None]

    w1f, wrf, w2s, w3s = [], [], [], []
    b1s, b2s, b3s, brs, lgs, lbs = [], [], [], [], [], []
    for name, (a, b) in zip(PART_NAMES, PART_SLICES):
        ep = p["feat_experts"][name]
        w1f.append(jnp.zeros((FEAT_DIM, EH), f32).at[a:b].set(ep["w1"]))
        wrf.append(jnp.zeros((FEAT_DIM, EO), f32).at[a:b].set(ep["wr"]))
        w2s.append(ep["w2"])
        w3s.append(ep["w3"])
        b1s.append(ep["b1"][None])
        b2s.append(ep["b2"][None])
        b3s.append(ep["b3"][None])
        brs.append(ep["br"][None])
        lgs.append(ep["lng"][None])
        lbs.append(ep["lnb"][None])
    w["fe_w1"] = jnp.stack(w1f)
    w["fe_wr"] = jnp.stack(wrf)
    w["fe_w2"] = jnp.stack(w2s)
    w["fe_w3"] = jnp.stack(w3s)
    w["fe_b1"] = jnp.stack(b1s)
    w["fe_b2"] = jnp.stack(b2s)
    w["fe_b3"] = jnp.stack(b3s)
    w["fe_br"] = jnp.stack(brs)
    w["fe_lng"] = jnp.stack(lgs)
    w["fe_lnb"] = jnp.stack(lbs)

    cw = p["context"]["w"]
    w["ctx_wa"] = cw[0:N_ACC]
    w["ctx_wc"] = cw[N_ACC:N_ACC + COIN_DIM]
    w["ctx_wr"] = cw[N_ACC + COIN_DIM:N_ACC + COIN_DIM + REG_DIM]
    w["ctx_wt"] = cw[N_ACC + COIN_DIM + REG_DIM:]
    w["ctx_b"] = p["context"]["b"][None]

    qw = p["gate_q"]["w"]
    w["gq_parts"] = jnp.stack([qw[i * EO:(i + 1) * EO] for i in range(4)])
    w["gq_ctx"] = qw[4 * EO:]
    w["gq_b"] = p["gate_q"]["b"][None]
    w["gk_w"] = jnp.stack([p["gate_keys"][n]["w"] for n in PART_NAMES])
    w["gk_b"] = jnp.stack([p["gate_keys"][n]["b"][None] for n in PART_NAMES])

    rw = p["router1"]["w"]
    w["r1_g"] = rw[:EO]
    w["r1_r"] = rw[EO:]
    w["r1_b"] = p["router1"]["b"][None]
    w["r2_w"] = p["router2"]["w"]
    w["r2_b"] = p["router2"]["b"][None]

    w["moe_w1"] = jnp.stack([e["w1"] for e in p["moe_experts"]])
    w["moe_w2"] = jnp.stack([e["w2"] for e in p["moe_experts"]])
    w["moe_w3"] = jnp.stack([e["w3"] for e in p["moe_experts"]])
    w["moe_b1"] = jnp.stack([e["b1"][None] for e in p["moe_experts"]])
    w["moe_b2"] = jnp.stack([e["b2"][None] for e in p["moe_experts"]])
    w["moe_b3"] = jnp.stack([e["b3"][None] for e in p["moe_experts"]])
    w["moe_lng"] = jnp.stack([e["lng"][None] for e in p["moe_experts"]])
    w["moe_lnb"] = jnp.stack([e["lnb"][None] for e in p["moe_experts"]])

    fw = p["fus1"]["w"]
    w["f1_m"] = fw[:EO]
    w["f1_c"] = fw[EO:]
    w["f1_b"] = p["fus1"]["b"][None]
    w["f_ln1g"] = p["fus_ln1g"][None]
    w["f_ln1b"] = p["fus_ln1b"][None]
    w["f2_w"] = p["fus2"]["w"]
    w["f2_b"] = p["fus2"]["b"][None]
    w["f_ln2g"] = p["fus_ln2g"][None]
    w["f_ln2b"] = p["fus_ln2b"][None]

    for hname in ("lab", "mae", "mfe"):
        for lyr in ("1", "2"):
            key = hname + lyr
            w[f"hd_{key}_w"] = jnp.stack([h[key]["w"] for h in p["heads"]])
            w[f"hd_{key}_b"] = jnp.stack([h[key]["b"][None] for h in p["heads"]])
    w["conf1_w"] = p["conf1"]["w"]
    w["conf1_b"] = p["conf1"]["b"][None]
    w["conf2_w"] = p["conf2"]["w"]
    w["conf2_b"] = p["conf2"]["b"][None]
    w["lev1_w"] = p["lev1"]["w"]
    w["lev1_b"] = p["lev1"]["b"][None]
    w["lev2_w"] = p["lev2"]["w"]
    w["lev2_b"] = p["lev2"]["b"][None]
    return w


_WNAMES = None


def _body(names, *refs):
    feats_ref, coin_ref, reg_ref, acct_ref, temp_ref = refs[:5]
    out_ref = refs[-1]
    w = {n: r for n, r in zip(names, refs[5:-1])}

    feats = feats_ref[...]
    coin_id = coin_ref[...]
    regime_id = reg_ref[...]
    acct = acct_ref[...]
    temporal = temp_ref[...]

    iota_c = lax.broadcasted_iota(jnp.int32, (BLK, 256), 1)
    oh_c = (iota_c == coin_id).astype(jnp.float32)
    coin_emb = _dot(oh_c, w["coin_emb"][...])
    iota_r = lax.broadcasted_iota(jnp.int32, (BLK, 128), 1)
    oh_r = (iota_r == regime_id).astype(jnp.float32)
    regime_emb = _dot(oh_r, w["regime_emb"][...])

    t = _gelu(_dot(temporal, w["temp1_w"][...]) + w["temp1_b"][...])
    t = _dot(t, w["temp2_w"][...]) + w["temp2_b"][...]
    temporal_enc = _ln(t, w["temp_lng"][...], w["temp_lnb"][...])

    feat_outs = []
    for i in range(4):
        h = _gelu(_dot(feats, w["fe_w1"][i]) + w["fe_b1"][i])
        h = _gelu(_dot(h, w["fe_w2"][i]) + w["fe_b2"][i])
        h = _dot(h, w["fe_w3"][i]) + w["fe_b3"][i]
        res = _dot(feats, w["fe_wr"][i]) + w["fe_br"][i]
        feat_outs.append(_ln(h + res, w["fe_lng"][i], w["fe_lnb"][i]))

    ctx = (_dot(acct, w["ctx_wa"][...]) + _dot(coin_emb, w["ctx_wc"][...])
           + _dot(regime_emb, w["ctx_wr"][...])
           + _dot(temporal_enc, w["ctx_wt"][...]) + w["ctx_b"][...])
    context_enc = _gelu(ctx)

    q = w["gq_b"][...] + _dot(context_enc, w["gq_ctx"][...])
    for i in range(4):
        q = q + _dot(feat_outs[i], w["gq_parts"][i])
    scores = []
    for i in range(4):
        k = _dot(feat_outs[i], w["gk_w"][i]) + w["gk_b"][i]
        scores.append(jnp.sum(q * k, axis=-1, keepdims=True) * _RSQRT_EO)
    smax = jnp.maximum(jnp.maximum(scores[0], scores[1]),
                       jnp.maximum(scores[2], scores[3]))
    exps = [jnp.exp(s - smax) for s in scores]
    denom = exps[0] + exps[1] + exps[2] + exps[3]
    gated = jnp.zeros((BLK, EO), jnp.float32)
    for i in range(4):
        gated = gated + (exps[i] / denom) * feat_outs[i]

    rh = _gelu(_dot(gated, w["r1_g"][...]) + _dot(regime_emb, w["r1_r"][...])
               + w["r1_b"][...])
    logits = _dot(rh, w["r2_w"][...]) + w["r2_b"][...]
    iota8 = lax.broadcasted_iota(jnp.int32, (BLK, NE), 1)
    m1 = jnp.max(logits, axis=-1, keepdims=True)
    i1 = jnp.min(jnp.where(logits == m1, iota8, NE), axis=-1, keepdims=True)
    masked = jnp.where(iota8 == i1, -1e30, logits)
    m2 = jnp.max(masked, axis=-1, keepdims=True)
    i2 = jnp.min(jnp.where(masked == m2, iota8, NE), axis=-1, keepdims=True)
    e2 = jnp.exp(m2 - m1)
    w1c = 1.0 / (1.0 + e2)
    w2c = e2 * w1c
    coefs = (jnp.where(iota8 == i1, w1c, 0.0)
             + jnp.where(iota8 == i2, w2c, 0.0))

    moe = jnp.zeros((BLK, EO), jnp.float32)
    for e in range(NE):
        h = _gelu(_dot(gated, w["moe_w1"][e]) + w["moe_b1"][e])
        h = _gelu(_dot(h, w["moe_w2"][e]) + w["moe_b2"][e])
        h = _dot(h, w["moe_w3"][e]) + w["moe_b3"][e]
        eo = _ln(h + gated, w["moe_lng"][e], w["moe_lnb"][e])
        moe = moe + lax.slice_in_dim(coefs, e, e + 1, axis=1) * eo

    f = _gelu(_dot(moe, w["f1_m"][...]) + _dot(context_enc, w["f1_c"][...])
              + w["f1_b"][...])
    f = _ln(f, w["f_ln1g"][...], w["f_ln1b"][...])
    f = _gelu(_dot(f, w["f2_w"][...]) + w["f2_b"][...])
    f = _ln(f, w["f_ln2g"][...], w["f_ln2b"][...])

    pieces = []
    for hname in ("lab", "mae", "mfe"):
        for g in range(NGROUPS):
            h1 = _gelu(_dot(f, w[f"hd_{hname}1_w"][g]) + w[f"hd_{hname}1_b"][g])
            pieces.append(_dot(h1, w[f"hd_{hname}2_w"][g]) + w[f"hd_{hname}2_b"][g])
    c = _gelu(_dot(f, w["conf1_w"][...]) + w["conf1_b"][...])
    pieces.append(jax.nn.sigmoid(_dot(c, w["conf2_w"][...]) + w["conf2_b"][...]))
    lv = _gelu(_dot(f, w["lev1_w"][...]) + w["lev1_b"][...])
    pieces.append(jax.nn.sigmoid(_dot(lv, w["lev2_w"][...]) + w["lev2_b"][...]))
    out_ref[...] = jnp.concatenate(pieces, axis=-1)


def _forward(features, coin_id, regime_id, account, temporal, params,
             interpret=False):
    w = _pack_weights(params)
    names = tuple(w.keys())
    warrs = [w[n] for n in names]
    coin2 = coin_id.astype(jnp.int32).reshape(B, 1)
    reg2 = regime_id.astype(jnp.int32).reshape(B, 1)

    def _const_spec(arr):
        nd = arr.ndim
        return pl.BlockSpec(arr.shape, lambda i, _nd=nd: (0,) * _nd)

    in_specs = [
        pl.BlockSpec((BLK, FEAT_DIM), lambda i: (i, 0)),
        pl.BlockSpec((BLK, 1), lambda i: (i, 0)),
        pl.BlockSpec((BLK, 1), lambda i: (i, 0)),
        pl.BlockSpec((BLK, N_ACC), lambda i: (i, 0)),
        pl.BlockSpec((BLK, N_TEMP), lambda i: (i, 0)),
    ] + [_const_spec(a) for a in warrs]

    out = pl.pallas_call(
        functools.partial(_body, names),
        grid=(B // BLK,),
        in_specs=in_specs,
        out_specs=pl.BlockSpec((BLK, 98), lambda i: (i, 0)),
        out_shape=jax.ShapeDtypeStruct((B, 98), jnp.float32),
        interpret=interpret,
    )(features, coin2, reg2, account, temporal, *warrs)
    return out


def kernel(features, coin_id, regime_id, account, temporal, params):
    return _forward(features, coin_id, regime_id, account, temporal, params)


# speedup vs baseline: 2.6790x; 1.0522x over previous
import functools

import jax
import jax.numpy as jnp
from jax import lax
from jax.experimental import pallas as pl

B = 4096
BLK = 1024
N_COINS = 250
COIN_DIM = 32
REG_DIM = 16
N_ACC = 4
N_TEMP = 40
EH = 256
EO = 128
NE = 8
FUSION = 256
NLAB = 8
NGROUPS = 4
FEAT_DIM = 256
PART_SLICES = ((0, 64), (64, 128), (128, 192), (192, 256))
PART_NAMES = ("price", "volume", "orderflow", "derived")

_SQRT2 = 1.4142135623730951
_RSQRT_EO = 1.0 / (EO ** 0.5)


def _gelu(x):
    return 0.5 * x * (1.0 + lax.erf(x / _SQRT2))


def _ln(x, g, b, eps=1e-5):
    m = jnp.mean(x, axis=-1, keepdims=True)
    xc = x - m
    v = jnp.mean(xc * xc, axis=-1, keepdims=True)
    return xc * lax.rsqrt(v + eps) * g + b


def _dot(x, w):
    return jnp.dot(x, w, preferred_element_type=jnp.float32)


def _pack_weights(p):
    f32 = jnp.float32
    w = {}
    coin = jnp.zeros((256, COIN_DIM), f32).at[:N_COINS].set(p["coin_emb"])
    w["coin_emb"] = coin
    reg = jnp.zeros((128, REG_DIM), f32).at[:4].set(p["regime_emb"])
    w["regime_emb"] = reg
    w["temp1_w"] = p["temp1"]["w"]
    w["temp1_b"] = p["temp1"]["b"][None]
    w["temp2_w"] = p["temp2"]["w"]
    w["temp2_b"] = p["temp2"]["b"][None]
    w["temp_lng"] = p["temp_lng"][None]
    w["temp_lnb"] = p["temp_lnb"][None]

    w1f, wrf, w2s, w3s = [], [], [], []
    b1s, b2s, b3s, brs, lgs, lbs = [], [], [], [], [], []
    for name, (a, b) in zip(PART_NAMES, PART_SLICES):
        ep = p["feat_experts"][name]
        w1f.append(jnp.zeros((FEAT_DIM, EH), f32).at[a:b].set(ep["w1"]))
        wrf.append(jnp.zeros((FEAT_DIM, EO), f32).at[a:b].set(ep["wr"]))
        w2s.append(ep["w2"])
        w3s.append(ep["w3"])
        b1s.append(ep["b1"][None])
        b2s.append(ep["b2"][None])
        b3s.append(ep["b3"][None])
        brs.append(ep["br"][None])
        lgs.append(ep["lng"][None])
        lbs.append(ep["lnb"][None])
    w["fe_w1"] = jnp.stack(w1f)
    w["fe_wr"] = jnp.stack(wrf)
    w["fe_w2"] = jnp.stack(w2s)
    w["fe_w3"] = jnp.stack(w3s)
    w["fe_b1"] = jnp.stack(b1s)
    w["fe_b2"] = jnp.stack(b2s)
    w["fe_b3"] = jnp.stack(b3s)
    w["fe_br"] = jnp.stack(brs)
    w["fe_lng"] = jnp.stack(lgs)
    w["fe_lnb"] = jnp.stack(lbs)

    cw = p["context"]["w"]
    w["ctx_wa"] = cw[0:N_ACC]
    w["ctx_wc"] = cw[N_ACC:N_ACC + COIN_DIM]
    w["ctx_wr"] = cw[N_ACC + COIN_DIM:N_ACC + COIN_DIM + REG_DIM]
    w["ctx_wt"] = cw[N_ACC + COIN_DIM + REG_DIM:]
    w["ctx_b"] = p["context"]["b"][None]

    qw = p["gate_q"]["w"]
    w["gq_parts"] = jnp.stack([qw[i * EO:(i + 1) * EO] for i in range(4)])
    w["gq_ctx"] = qw[4 * EO:]
    w["gq_b"] = p["gate_q"]["b"][None]
    w["gk_w"] = jnp.stack([p["gate_keys"][n]["w"] for n in PART_NAMES])
    w["gk_b"] = jnp.stack([p["gate_keys"][n]["b"][None] for n in PART_NAMES])

    rw = p["router1"]["w"]
    w["r1_g"] = rw[:EO]
    w["r1_r"] = rw[EO:]
    w["r1_b"] = p["router1"]["b"][None]
    w["r2_w"] = p["router2"]["w"]
    w["r2_b"] = p["router2"]["b"][None]

    w["moe_w1"] = jnp.stack([e["w1"] for e in p["moe_experts"]])
    w["moe_w2"] = jnp.stack([e["w2"] for e in p["moe_experts"]])
    w["moe_w3"] = jnp.stack([e["w3"] for e in p["moe_experts"]])
    w["moe_b1"] = jnp.stack([e["b1"][None] for e in p["moe_experts"]])
    w["moe_b2"] = jnp.stack([e["b2"][None] for e in p["moe_experts"]])
    w["moe_b3"] = jnp.stack([e["b3"][None] for e in p["moe_experts"]])
    w["moe_lng"] = jnp.stack([e["lng"][None] for e in p["moe_experts"]])
    w["moe_lnb"] = jnp.stack([e["lnb"][None] for e in p["moe_experts"]])

    fw = p["fus1"]["w"]
    w["f1_m"] = fw[:EO]
    w["f1_c"] = fw[EO:]
    w["f1_b"] = p["fus1"]["b"][None]
    w["f_ln1g"] = p["fus_ln1g"][None]
    w["f_ln1b"] = p["fus_ln1b"][None]
    w["f2_w"] = p["fus2"]["w"]
    w["f2_b"] = p["fus2"]["b"][None]
    w["f_ln2g"] = p["fus_ln2g"][None]
    w["f_ln2b"] = p["fus_ln2b"][None]

    for hname in ("lab", "mae", "mfe"):
        for lyr in ("1", "2"):
            key = hname + lyr
            w[f"hd_{key}_w"] = jnp.stack([h[key]["w"] for h in p["heads"]])
            w[f"hd_{key}_b"] = jnp.stack([h[key]["b"][None] for h in p["heads"]])
    w["conf1_w"] = p["conf1"]["w"]
    w["conf1_b"] = p["conf1"]["b"][None]
    w["conf2_w"] = p["conf2"]["w"]
    w["conf2_b"] = p["conf2"]["b"][None]
    w["lev1_w"] = p["lev1"]["w"]
    w["lev1_b"] = p["lev1"]["b"][None]
    w["lev2_w"] = p["lev2"]["w"]
    w["lev2_b"] = p["lev2"]["b"][None]
    return w


_WNAMES = None


def _body(names, *refs):
    feats_ref, coin_ref, reg_ref, acct_ref, temp_ref = refs[:5]
    out_ref = refs[-1]
    w = {n: r for n, r in zip(names, refs[5:-1])}

    feats = feats_ref[...]
    coin_id = coin_ref[...]
    regime_id = reg_ref[...]
    acct = acct_ref[...]
    temporal = temp_ref[...]

    iota_c = lax.broadcasted_iota(jnp.int32, (BLK, 256), 1)
    oh_c = (iota_c == coin_id).astype(jnp.float32)
    coin_emb = _dot(oh_c, w["coin_emb"][...])
    iota_r = lax.broadcasted_iota(jnp.int32, (BLK, 128), 1)
    oh_r = (iota_r == regime_id).astype(jnp.float32)
    regime_emb = _dot(oh_r, w["regime_emb"][...])

    t = _gelu(_dot(temporal, w["temp1_w"][...]) + w["temp1_b"][...])
    t = _dot(t, w["temp2_w"][...]) + w["temp2_b"][...]
    temporal_enc = _ln(t, w["temp_lng"][...], w["temp_lnb"][...])

    feat_outs = []
    for i in range(4):
        h = _gelu(_dot(feats, w["fe_w1"][i]) + w["fe_b1"][i])
        h = _gelu(_dot(h, w["fe_w2"][i]) + w["fe_b2"][i])
        h = _dot(h, w["fe_w3"][i]) + w["fe_b3"][i]
        res = _dot(feats, w["fe_wr"][i]) + w["fe_br"][i]
        feat_outs.append(_ln(h + res, w["fe_lng"][i], w["fe_lnb"][i]))

    ctx = (_dot(acct, w["ctx_wa"][...]) + _dot(coin_emb, w["ctx_wc"][...])
           + _dot(regime_emb, w["ctx_wr"][...])
           + _dot(temporal_enc, w["ctx_wt"][...]) + w["ctx_b"][...])
    context_enc = _gelu(ctx)

    q = w["gq_b"][...] + _dot(context_enc, w["gq_ctx"][...])
    for i in range(4):
        q = q + _dot(feat_outs[i], w["gq_parts"][i])
    scores = []
    for i in range(4):
        k = _dot(feat_outs[i], w["gk_w"][i]) + w["gk_b"][i]
        scores.append(jnp.sum(q * k, axis=-1, keepdims=True) * _RSQRT_EO)
    smax = jnp.maximum(jnp.maximum(scores[0], scores[1]),
                       jnp.maximum(scores[2], scores[3]))
    exps = [jnp.exp(s - smax) for s in scores]
    denom = exps[0] + exps[1] + exps[2] + exps[3]
    gated = jnp.zeros((BLK, EO), jnp.float32)
    for i in range(4):
        gated = gated + (exps[i] / denom) * feat_outs[i]

    rh = _gelu(_dot(gated, w["r1_g"][...]) + _dot(regime_emb, w["r1_r"][...])
               + w["r1_b"][...])
    logits = _dot(rh, w["r2_w"][...]) + w["r2_b"][...]
    iota8 = lax.broadcasted_iota(jnp.int32, (BLK, NE), 1)
    m1 = jnp.max(logits, axis=-1, keepdims=True)
    i1 = jnp.min(jnp.where(logits == m1, iota8, NE), axis=-1, keepdims=True)
    masked = jnp.where(iota8 == i1, -1e30, logits)
    m2 = jnp.max(masked, axis=-1, keepdims=True)
    i2 = jnp.min(jnp.where(masked == m2, iota8, NE), axis=-1, keepdims=True)
    e2 = jnp.exp(m2 - m1)
    w1c = 1.0 / (1.0 + e2)
    w2c = e2 * w1c
    coefs = (jnp.where(iota8 == i1, w1c, 0.0)
             + jnp.where(iota8 == i2, w2c, 0.0))

    moe = jnp.zeros((BLK, EO), jnp.float32)
    for e in range(NE):
        h = _gelu(_dot(gated, w["moe_w1"][e]) + w["moe_b1"][e])
        h = _gelu(_dot(h, w["moe_w2"][e]) + w["moe_b2"][e])
        h = _dot(h, w["moe_w3"][e]) + w["moe_b3"][e]
        eo = _ln(h + gated, w["moe_lng"][e], w["moe_lnb"][e])
        moe = moe + lax.slice_in_dim(coefs, e, e + 1, axis=1) * eo

    f = _gelu(_dot(moe, w["f1_m"][...]) + _dot(context_enc, w["f1_c"][...])
              + w["f1_b"][...])
    f = _ln(f, w["f_ln1g"][...], w["f_ln1b"][...])
    f = _gelu(_dot(f, w["f2_w"][...]) + w["f2_b"][...])
    f = _ln(f, w["f_ln2g"][...], w["f_ln2b"][...])

    pieces = []
    for hname in ("lab", "mae", "mfe"):
        for g in range(NGROUPS):
            h1 = _gelu(_dot(f, w[f"hd_{hname}1_w"][g]) + w[f"hd_{hname}1_b"][g])
            pieces.append(_dot(h1, w[f"hd_{hname}2_w"][g]) + w[f"hd_{hname}2_b"][g])
    c = _gelu(_dot(f, w["conf1_w"][...]) + w["conf1_b"][...])
    pieces.append(jax.nn.sigmoid(_dot(c, w["conf2_w"][...]) + w["conf2_b"][...]))
    lv = _gelu(_dot(f, w["lev1_w"][...]) + w["lev1_b"][...])
    pieces.append(jax.nn.sigmoid(_dot(lv, w["lev2_w"][...]) + w["lev2_b"][...]))
    out_ref[...] = jnp.concatenate(pieces, axis=-1)


def _forward(features, coin_id, regime_id, account, temporal, params,
             interpret=False):
    w = _pack_weights(params)
    names = tuple(w.keys())
    warrs = [w[n] for n in names]
    coin2 = coin_id.astype(jnp.int32).reshape(B, 1)
    reg2 = regime_id.astype(jnp.int32).reshape(B, 1)

    def _const_spec(arr):
        nd = arr.ndim
        return pl.BlockSpec(arr.shape, lambda i, _nd=nd: (0,) * _nd)

    in_specs = [
        pl.BlockSpec((BLK, FEAT_DIM), lambda i: (i, 0)),
        pl.BlockSpec((BLK, 1), lambda i: (i, 0)),
        pl.BlockSpec((BLK, 1), lambda i: (i, 0)),
        pl.BlockSpec((BLK, N_ACC), lambda i: (i, 0)),
        pl.BlockSpec((BLK, N_TEMP), lambda i: (i, 0)),
    ] + [_const_spec(a) for a in warrs]

    out = pl.pallas_call(
        functools.partial(_body, names),
        grid=(B // BLK,),
        in_specs=in_specs,
        out_specs=pl.BlockSpec((BLK, 98), lambda i: (i, 0)),
        out_shape=jax.ShapeDtypeStruct((B, 98), jnp.float32),
        interpret=interpret,
    )(features, coin2, reg2, account, temporal, *warrs)
    return out


def kernel(features, coin_id, regime_id, account, temporal, params):
    return _forward(features, coin_id, regime_id, account, temporal, params)
